# baseline (speedup 1.0000x reference)
import functools

import jax
import jax.numpy as jnp
from jax.experimental import pallas as pl
from jax.experimental.pallas import tpu as pltpu


def _concat_body(n, r_ref, g_ref, o_ref):
    o_ref[:, :, :n] = r_ref[...]
    o_ref[:, :, n:] = g_ref[...]


def kernel(rois, gt_boxes):
    B, N, C = rois.shape
    _, G, _ = gt_boxes.shape
    r_t = jnp.transpose(rois, (0, 2, 1))
    g_t = jnp.transpose(gt_boxes, (0, 2, 1))
    body = functools.partial(_concat_body, N)
    out_t = pl.pallas_call(
        body,
        grid=(B,),
        in_specs=[
            pl.BlockSpec((1, C, N), lambda i: (i, 0, 0)),
            pl.BlockSpec((1, C, G), lambda i: (i, 0, 0)),
        ],
        out_specs=pl.BlockSpec((1, C, N + G), lambda i: (i, 0, 0)),
        out_shape=jax.ShapeDtypeStruct((B, C, N + G), rois.dtype),
        compiler_params=pltpu.CompilerParams(
            dimension_semantics=("arbitrary",),
            skip_device_barrier=True,
            disable_bounds_checks=True,
        ),
    )(r_t, g_t)
    return jnp.transpose(out_t, (0, 2, 1))
